# baseline (speedup 1.0000x reference)
from functools import partial

import jax
import jax.numpy as jnp
from jax import lax
from jax.experimental import pallas as pl
from jax.experimental.pallas import tpu as pltpu
from jax.experimental.pallas import tpu_sc as plsc

_E = 8
_K = 2
_T = 4096
_NW = 32
_TW = 1024
_NEG = -3.0e38


def _tc_scores(x_ref, w_ref, s_ref):
    xb = x_ref[...]
    w = w_ref[...]
    scores = jax.lax.dot_general(
        w, xb, (((1,), (1,)), ((), ())),
        preferred_element_type=jnp.float32)
    s_ref[...] = scores.reshape(_E, _T // _TW, _TW).swapaxes(0, 1)


def _sc_route(s_hbm, idx_hbm, wgt_hbm, usage_hbm, sv, oi, ow, uv):
    wid = lax.axis_index("s") * 2 + lax.axis_index("c")
    pltpu.sync_copy(s_hbm.at[wid], sv)

    accs = [jnp.zeros((16,), jnp.float32) for _ in range(_E)]
    for g in range(_TW // 16):
        sl = pl.ds(g * 16, 16)
        s = [sv[e, sl] for e in range(_E)]
        m1 = s[0]
        for e in range(1, _E):
            m1 = jnp.maximum(m1, s[e])
        i1 = jnp.zeros((16,), jnp.int32)
        for e in range(_E - 1, -1, -1):
            i1 = jnp.where(s[e] == m1, jnp.int32(e), i1)
        cand = [jnp.where(i1 == e, jnp.float32(_NEG), s[e]) for e in range(_E)]
        m2 = cand[0]
        for e in range(1, _E):
            m2 = jnp.maximum(m2, cand[e])
        i2 = jnp.zeros((16,), jnp.int32)
        for e in range(_E - 1, -1, -1):
            i2 = jnp.where(cand[e] == m2, jnp.int32(e), i2)

        ee = jnp.exp(m2 - m1)
        w1 = 1.0 / (1.0 + ee)

        psum = None
        ps = []
        for e in range(_E):
            p = jnp.exp(s[e] - m1)
            ps.append(p)
            psum = p if psum is None else psum + p
        inv = 1.0 / psum
        for e in range(_E):
            accs[e] = accs[e] + ps[e] * inv

        oi[0, sl] = i1
        oi[1, sl] = i2
        ow[0, sl] = w1
        ow[1, sl] = 1.0 - w1

    for e in range(_E):
        uv[e, :] = accs[e]

    pltpu.sync_copy(oi, idx_hbm.at[wid])
    pltpu.sync_copy(ow, wgt_hbm.at[wid])
    pltpu.sync_copy(uv, usage_hbm.at[wid])


@jax.jit
def kernel(x, W):
    B, S, D = x.shape
    N = B * S
    xf = x.reshape(N, D)
    nsteps = N // _T
    wps = _T // _TW

    scores = pl.pallas_call(
        _tc_scores,
        grid=(nsteps,),
        in_specs=[
            pl.BlockSpec((_T, D), lambda i: (i, 0)),
            pl.BlockSpec((_E, D), lambda i: (0, 0)),
        ],
        out_specs=pl.BlockSpec((wps, _E, _TW), lambda i: (i, 0, 0)),
        out_shape=jax.ShapeDtypeStruct((_NW, _E, _TW), jnp.float32),
    )(xf, W)

    mesh = plsc.VectorSubcoreMesh(core_axis_name="c", subcore_axis_name="s")
    idx_h, wgt_h, usage_h = pl.kernel(
        _sc_route,
        out_type=[
            jax.ShapeDtypeStruct((_NW, _K, _TW), jnp.int32),
            jax.ShapeDtypeStruct((_NW, _K, _TW), jnp.float32),
            jax.ShapeDtypeStruct((_NW, _E, 16), jnp.float32),
        ],
        mesh=mesh,
        scratch_types=[
            pltpu.VMEM((_E, _TW), jnp.float32),
            pltpu.VMEM((_K, _TW), jnp.int32),
            pltpu.VMEM((_K, _TW), jnp.float32),
            pltpu.VMEM((_E, 16), jnp.float32),
        ],
    )(scores)

    idx = idx_h.transpose(0, 2, 1).reshape(B, S, _K)
    wgt = wgt_h.transpose(0, 2, 1).reshape(B, S, _K)
    usage = usage_h.sum(axis=(0, 2)) / N
    aux = _E * jnp.sum(usage * usage)
    return idx, wgt, aux
